# baseline (speedup 1.0000x reference)
import functools
import jax
import jax.numpy as jnp
from jax import lax
from jax.experimental import pallas as pl
from jax.experimental.pallas import tpu as pltpu
from jax.experimental.pallas import tpu_sc as plsc

NC = 2
NS = 16
NW = NC * NS

D_MODEL = 2048
N_ROWS = 4
ROW_LEN = 4096
B_PER_W = N_ROWS * ROW_LEN // NW
W_PER_ROW = ROW_LEN // B_PER_W
CHUNK = 16
N_CHUNKS = B_PER_W // CHUNK
NB = 3


def _make_gather():
  mesh = plsc.VectorSubcoreMesh(
      core_axis_name="c", subcore_axis_name="s",
      num_cores=NC, num_subcores=NS)

  @functools.partial(
      pl.kernel,
      out_type=jax.ShapeDtypeStruct((N_ROWS, ROW_LEN, D_MODEL),
                                    jnp.float32),
      mesh=mesh,
      scratch_types=[
          pltpu.VMEM((B_PER_W,), jnp.int32),
          pltpu.VMEM((NB, CHUNK, D_MODEL), jnp.float32),
          pltpu.SemaphoreType.DMA((NB,)),
          pltpu.SemaphoreType.DMA((NB,)),
      ],
  )
  def gather_kernel(idx_hbm, table_hbm, out_hbm, idx_v, bufs, gsem, wsem):
    wid = lax.axis_index("s") * NC + lax.axis_index("c")
    row = wid // W_PER_ROW
    col0 = (wid % W_PER_ROW) * B_PER_W
    pltpu.sync_copy(idx_hbm.at[row, pl.ds(col0, B_PER_W)], idx_v)

    def gather(c, b):
      return pltpu.make_async_copy(
          table_hbm.at[idx_v.at[pl.ds(c * CHUNK, CHUNK)]],
          bufs.at[b], gsem.at[b])

    def write(c, b):
      return pltpu.make_async_copy(
          bufs.at[b], out_hbm.at[row, pl.ds(col0 + c * CHUNK, CHUNK)],
          wsem.at[b])

    for c in range(NB):
      gather(c, c % NB).start()

    for c in range(N_CHUNKS):
      b = c % NB
      gather(c, b).wait()
      write(c, b).start()
      cn = c + NB - 1
      if NB <= cn < N_CHUNKS:
        write(cn - NB, cn % NB).wait()
        gather(cn, cn % NB).start()

    for c in range(N_CHUNKS - NB, N_CHUNKS):
      write(c, c % NB).wait()

  return gather_kernel


_gather = _make_gather()


@jax.jit
def kernel(tokens, W_E):
  return _gather(tokens.astype(jnp.int32), W_E)


# speedup vs baseline: 1.0053x; 1.0053x over previous
import functools
import jax
import jax.numpy as jnp
from jax import lax
from jax.experimental import pallas as pl
from jax.experimental.pallas import tpu as pltpu
from jax.experimental.pallas import tpu_sc as plsc

NC = 2
NS = 16
NW = NC * NS

D_MODEL = 2048
N_ROWS = 4
ROW_LEN = 4096
W_PER_ROW = ROW_LEN // (ROW_LEN * N_ROWS // NW)
B_PER_W = N_ROWS * ROW_LEN // NW
CHUNK = 16
N_CHUNKS = B_PER_W // CHUNK
NB = 2


def _make_gather():
  mesh = plsc.VectorSubcoreMesh(
      core_axis_name="c", subcore_axis_name="s",
      num_cores=NC, num_subcores=NS)

  @functools.partial(
      pl.kernel,
      out_type=jax.ShapeDtypeStruct((N_ROWS, ROW_LEN, D_MODEL),
                                    jnp.float32),
      mesh=mesh,
      scratch_types=[
          pltpu.VMEM((B_PER_W,), jnp.int32),
          pltpu.VMEM((NB, CHUNK, D_MODEL), jnp.float32),
          pltpu.SemaphoreType.DMA((NB,)),
          pltpu.SemaphoreType.DMA((NB,)),
      ],
  )
  def gather_kernel(idx_hbm, table_hbm, out_hbm, idx_v, bufs, gsem, wsem):
    wid = lax.axis_index("s") * NC + lax.axis_index("c")
    row = wid // W_PER_ROW
    col0 = (wid % W_PER_ROW) * B_PER_W
    pltpu.sync_copy(idx_hbm.at[row, pl.ds(col0, B_PER_W)], idx_v)

    def gather(c, b):
      return pltpu.make_async_copy(
          table_hbm.at[idx_v.at[pl.ds(c * CHUNK, CHUNK)]],
          bufs.at[b], gsem.at[b])

    def write(c, b):
      return pltpu.make_async_copy(
          bufs.at[b], out_hbm.at[row, pl.ds(col0 + c * CHUNK, CHUNK)],
          wsem.at[b])

    for b in range(NB):
      gather(b, b).start()

    @pl.loop(0, N_CHUNKS, step=NB)
    def _(j):
      for b in range(NB):
        c = j + b
        gather(c, b).wait()
        write(c, b).start()
        cn = c + NB - 1
        bn = (b + NB - 1) % NB

        @pl.when(jnp.logical_and(cn >= NB, cn < N_CHUNKS))
        def _():
          write(c - 1, bn).wait()
          gather(cn, bn).start()

    for b in range(NB):
      c = N_CHUNKS - NB + b
      write(c, c % NB).wait()

  return gather_kernel


_gather = _make_gather()


@jax.jit
def kernel(tokens, W_E):
  return _gather(tokens.astype(jnp.int32), W_E)


# speedup vs baseline: 1.0278x; 1.0224x over previous
import functools
import jax
import jax.numpy as jnp
from jax import lax
from jax.experimental import pallas as pl
from jax.experimental.pallas import tpu as pltpu
from jax.experimental.pallas import tpu_sc as plsc

NC = 2
NS = 16
NW = NC * NS

D_MODEL = 2048
N_ROWS = 4
ROW_LEN = 4096
B_PER_W = N_ROWS * ROW_LEN // NW
W_PER_ROW = ROW_LEN // B_PER_W
CHUNK = 16
N_CHUNKS = B_PER_W // CHUNK
NB = 2


def _make_gather():
  mesh = plsc.VectorSubcoreMesh(
      core_axis_name="c", subcore_axis_name="s",
      num_cores=NC, num_subcores=NS)

  @functools.partial(
      pl.kernel,
      out_type=jax.ShapeDtypeStruct((N_ROWS, ROW_LEN, D_MODEL),
                                    jnp.float32),
      mesh=mesh,
      scratch_types=[
          pltpu.VMEM((B_PER_W,), jnp.int32),
          pltpu.VMEM((NB, CHUNK, D_MODEL), jnp.float32),
          pltpu.SemaphoreType.DMA((NB,)),
      ],
  )
  def gather_kernel(idx_hbm, table_hbm, out_hbm, idx_v, bufs, gsem):
    wid = lax.axis_index("s") * NC + lax.axis_index("c")
    row = wid // W_PER_ROW
    col0 = (wid % W_PER_ROW) * B_PER_W
    pltpu.sync_copy(idx_hbm.at[row, pl.ds(col0, B_PER_W)], idx_v)

    def gather(c, b):
      return pltpu.make_async_copy(
          table_hbm.at[idx_v.at[pl.ds(c * CHUNK, CHUNK)]],
          bufs.at[b], gsem.at[b])

    for b in range(NB):
      gather(b, b).start()

    @pl.loop(0, N_CHUNKS, step=NB)
    def _(j):
      for b in range(NB):
        c = j + b
        gather(c, b).wait()
        pltpu.sync_copy(bufs.at[b],
                        out_hbm.at[row, pl.ds(col0 + c * CHUNK, CHUNK)])

        @pl.when(c + NB < N_CHUNKS)
        def _():
          gather(c + NB, b).start()

  return gather_kernel


_gather = _make_gather()


@jax.jit
def kernel(tokens, W_E):
  return _gather(tokens.astype(jnp.int32), W_E)


# speedup vs baseline: 1.0328x; 1.0049x over previous
import functools
import jax
import jax.numpy as jnp
from jax import lax
from jax.experimental import pallas as pl
from jax.experimental.pallas import tpu as pltpu
from jax.experimental.pallas import tpu_sc as plsc

NC = 2
NS = 16
NW = NC * NS

D_MODEL = 2048
N_ROWS = 4
ROW_LEN = 4096
B_PER_W = N_ROWS * ROW_LEN // NW
W_PER_ROW = ROW_LEN // B_PER_W
CHUNK = 16
N_CHUNKS = B_PER_W // CHUNK
NB = 3


def _make_gather():
  mesh = plsc.VectorSubcoreMesh(
      core_axis_name="c", subcore_axis_name="s",
      num_cores=NC, num_subcores=NS)

  @functools.partial(
      pl.kernel,
      out_type=jax.ShapeDtypeStruct((N_ROWS, ROW_LEN, D_MODEL),
                                    jnp.float32),
      mesh=mesh,
      scratch_types=[
          pltpu.VMEM((B_PER_W,), jnp.int32),
          pltpu.VMEM((NB, CHUNK, D_MODEL), jnp.float32),
          pltpu.SemaphoreType.DMA((NB,)),
      ],
  )
  def gather_kernel(idx_hbm, table_hbm, out_hbm, idx_v, bufs, gsem):
    wid = lax.axis_index("s") * NC + lax.axis_index("c")
    row = wid // W_PER_ROW
    col0 = (wid % W_PER_ROW) * B_PER_W
    pltpu.sync_copy(idx_hbm.at[row, pl.ds(col0, B_PER_W)], idx_v)

    def gather(c, b):
      return pltpu.make_async_copy(
          table_hbm.at[idx_v.at[pl.ds(c * CHUNK, CHUNK)]],
          bufs.at[b], gsem.at[b])

    for b in range(NB):
      gather(b, b).start()

    @pl.loop(0, N_CHUNKS)
    def _(c):
      b = lax.rem(c, NB)
      gather(c, b).wait()
      pltpu.sync_copy(bufs.at[b],
                      out_hbm.at[row, pl.ds(col0 + c * CHUNK, CHUNK)])

      @pl.when(c + NB < N_CHUNKS)
      def _():
        gather(c + NB, b).start()

  return gather_kernel


_gather = _make_gather()


@jax.jit
def kernel(tokens, W_E):
  return _gather(tokens.astype(jnp.int32), W_E)


# speedup vs baseline: 1.0354x; 1.0025x over previous
import functools
import jax
import jax.numpy as jnp
from jax import lax
from jax.experimental import pallas as pl
from jax.experimental.pallas import tpu as pltpu
from jax.experimental.pallas import tpu_sc as plsc

NC = 2
NS = 16
NW = NC * NS

D_MODEL = 2048
N_ROWS = 4
ROW_LEN = 4096
B_PER_W = N_ROWS * ROW_LEN // NW
W_PER_ROW = ROW_LEN // B_PER_W
CHUNK = 8
N_CHUNKS = B_PER_W // CHUNK
NB = 6


def _make_gather():
  mesh = plsc.VectorSubcoreMesh(
      core_axis_name="c", subcore_axis_name="s",
      num_cores=NC, num_subcores=NS)

  @functools.partial(
      pl.kernel,
      out_type=jax.ShapeDtypeStruct((N_ROWS, ROW_LEN, D_MODEL),
                                    jnp.float32),
      mesh=mesh,
      scratch_types=[
          pltpu.VMEM((B_PER_W,), jnp.int32),
          pltpu.VMEM((NB, CHUNK, D_MODEL), jnp.float32),
          pltpu.SemaphoreType.DMA((NB,)),
      ],
  )
  def gather_kernel(idx_hbm, table_hbm, out_hbm, idx_v, bufs, gsem):
    wid = lax.axis_index("s") * NC + lax.axis_index("c")
    row = wid // W_PER_ROW
    col0 = (wid % W_PER_ROW) * B_PER_W
    pltpu.sync_copy(idx_hbm.at[row, pl.ds(col0, B_PER_W)], idx_v)

    def gather(c, b):
      return pltpu.make_async_copy(
          table_hbm.at[idx_v.at[pl.ds(c * CHUNK, CHUNK)]],
          bufs.at[b], gsem.at[b])

    for b in range(NB):
      gather(b, b).start()

    @pl.loop(0, N_CHUNKS)
    def _(c):
      b = lax.rem(c, NB)
      gather(c, b).wait()
      pltpu.sync_copy(bufs.at[b],
                      out_hbm.at[row, pl.ds(col0 + c * CHUNK, CHUNK)])

      @pl.when(c + NB < N_CHUNKS)
      def _():
        gather(c + NB, b).start()

  return gather_kernel


_gather = _make_gather()


@jax.jit
def kernel(tokens, W_E):
  return _gather(tokens.astype(jnp.int32), W_E)


# speedup vs baseline: 1.0357x; 1.0003x over previous
import functools
import jax
import jax.numpy as jnp
from jax import lax
from jax.experimental import pallas as pl
from jax.experimental.pallas import tpu as pltpu
from jax.experimental.pallas import tpu_sc as plsc

NC = 2
NS = 16
NW = NC * NS

D_MODEL = 2048
N_ROWS = 4
ROW_LEN = 4096
B_PER_W = N_ROWS * ROW_LEN // NW
W_PER_ROW = ROW_LEN // B_PER_W
CHUNK = 8
N_CHUNKS = B_PER_W // CHUNK
NB = 7


def _make_gather():
  mesh = plsc.VectorSubcoreMesh(
      core_axis_name="c", subcore_axis_name="s",
      num_cores=NC, num_subcores=NS)

  @functools.partial(
      pl.kernel,
      out_type=jax.ShapeDtypeStruct((N_ROWS, ROW_LEN, D_MODEL),
                                    jnp.float32),
      mesh=mesh,
      scratch_types=[
          pltpu.VMEM((B_PER_W,), jnp.int32),
          pltpu.VMEM((NB, CHUNK, D_MODEL), jnp.float32),
          pltpu.SemaphoreType.DMA((NB,)),
      ],
  )
  def gather_kernel(idx_hbm, table_hbm, out_hbm, idx_v, bufs, gsem):
    wid = lax.axis_index("s") * NC + lax.axis_index("c")
    row = wid // W_PER_ROW
    col0 = (wid % W_PER_ROW) * B_PER_W
    pltpu.sync_copy(idx_hbm.at[row, pl.ds(col0, B_PER_W)], idx_v)

    def gather(c, b):
      return pltpu.make_async_copy(
          table_hbm.at[idx_v.at[pl.ds(c * CHUNK, CHUNK)]],
          bufs.at[b], gsem.at[b])

    for b in range(NB):
      gather(b, b).start()

    @pl.loop(0, N_CHUNKS)
    def _(c):
      b = lax.rem(c, NB)
      gather(c, b).wait()
      pltpu.sync_copy(bufs.at[b],
                      out_hbm.at[row, pl.ds(col0 + c * CHUNK, CHUNK)])

      @pl.when(c + NB < N_CHUNKS)
      def _():
        gather(c + NB, b).start()

  return gather_kernel


_gather = _make_gather()


@jax.jit
def kernel(tokens, W_E):
  return _gather(tokens.astype(jnp.int32), W_E)
